# Initial kernel scaffold; baseline (speedup 1.0000x reference)
import jax
import jax.numpy as jnp
from jax.experimental import pallas as pl


def kernel(u, edge_index, neighbor_all, emb_id, att_fc1_w, att_fc1_b, att_fc2_w, att_fc2_b, w, fc1_w, fc1_b, fc2_w, fc2_b):
    raise NotImplementedError("write your pallas kernel here")


# speedup vs baseline: 5.5680x; 5.5680x over previous
import functools

import jax
import jax.numpy as jnp
from jax import lax
from jax.experimental import pallas as pl
from jax.experimental.pallas import tpu as pltpu
from jax.experimental.pallas import tpu_sc as plsc

NC = 2
NS = 16
NW = NC * NS
L = 16

F32 = jnp.float32
I32 = jnp.int32


def _sc_mesh():
    return plsc.VectorSubcoreMesh(
        core_axis_name="c", subcore_axis_name="s", num_cores=NC, num_subcores=NS
    )


def _wid():
    return lax.axis_index("s") * NC + lax.axis_index("c")


def _k1_edge_features(uxb, k_idx, i_idx, E, H, C):
    N = uxb.shape[0]
    PW = E // NW
    NCH = PW // C
    H2 = uxb.shape[1]
    BF = jnp.bfloat16

    assert NCH % 2 == 0

    @functools.partial(
        pl.kernel,
        out_type=jax.ShapeDtypeStruct((E, H), F32),
        mesh=_sc_mesh(),
        scratch_types=[
            pltpu.VMEM_SHARED((N, H2), F32),
            pltpu.VMEM((C,), I32),
            pltpu.VMEM((C,), I32),
            pltpu.VMEM((C,), I32),
            pltpu.VMEM((C,), I32),
            pltpu.VMEM((C, H2), F32),
            pltpu.VMEM((C, H2), F32),
            pltpu.VMEM((C, H2), F32),
            pltpu.VMEM((C, H2), F32),
            pltpu.VMEM((C, H), F32),
            pltpu.VMEM((C, H), F32),
            pltpu.SemaphoreType.DMA,
            pltpu.SemaphoreType.DMA,
            pltpu.SemaphoreType.DMA,
            pltpu.SemaphoreType.DMA,
            pltpu.SemaphoreType.DMA,
            pltpu.SemaphoreType.DMA,
        ],
        compiler_params=pltpu.CompilerParams(needs_layout_passes=False),
    )
    def k1(ux_hbm, k_hbm, i_hbm, out_hbm,
           ux_sp, kb0, ib0, kb1, ib1, abuf0, bbuf0, abuf1, bbuf1,
           hbuf0, hbuf1, isem0, isem1, gsem0, gsem1, wsem0, wsem1):
        base = pl.multiple_of(_wid() * PW, 8)

        @pl.when(lax.axis_index("s") == 0)
        def _stage():
            pltpu.sync_copy(ux_hbm, ux_sp)

        plsc.subcore_barrier()

        def issue_idx(c, kb, ib, isem):
            g = pl.multiple_of(base + c * C, 8)
            pltpu.async_copy(k_hbm.at[pl.ds(g, C)], kb, isem)
            pltpu.async_copy(i_hbm.at[pl.ds(g, C)], ib, isem)

        def wait_idx(kb, ib, isem):
            pltpu.make_async_copy(k_hbm.at[pl.ds(0, C)], kb, isem).wait()
            pltpu.make_async_copy(i_hbm.at[pl.ds(0, C)], ib, isem).wait()

        def issue_gather(kb, ib, ab, bb, gsem):
            pltpu.async_copy(ux_sp.at[kb], ab, gsem)
            pltpu.async_copy(ux_sp.at[ib], bb, gsem)

        def wait_gather(ab, bb, gsem):
            pltpu.make_async_copy(ux_sp.at[pl.ds(0, C)], ab, gsem).wait()
            pltpu.make_async_copy(ux_sp.at[pl.ds(0, C)], bb, gsem).wait()

        def compute(ab, bb, hb):
            fmt = plsc.PackFormat.INTERLEAVED

            def up(ref, e, sl):
                return plsc.unpack(plsc.bitcast(ref[e, sl], BF), format=fmt)

            def edge(e, ecarry):
                for j in range(H // 32):
                    slu = pl.ds(L * j, L)
                    slx = pl.ds(H // 2 + L * j, L)
                    uk0, uk1 = up(ab, e, slu)
                    ui0, ui1 = up(bb, e, slu)
                    xk0, xk1 = up(ab, e, slx)
                    xi0, xi1 = up(bb, e, slx)
                    hb[e, pl.ds(32 * j, L)] = (uk0 - ui0) * (xk0 - xi0)
                    hb[e, pl.ds(32 * j + L, L)] = (uk1 - ui1) * (xk1 - xi1)
                return ecarry

            lax.fori_loop(0, C, edge, 0)

        def issue_write(c, hb, wsem):
            g = pl.multiple_of(base + c * C, 8)
            pltpu.async_copy(hb, out_hbm.at[pl.ds(g, C)], wsem)

        def wait_write(hb, wsem):
            pltpu.make_async_copy(hb, out_hbm.at[pl.ds(0, C)], wsem).wait()

        issue_idx(0, kb0, ib0, isem0)
        wait_idx(kb0, ib0, isem0)
        issue_gather(kb0, ib0, abuf0, bbuf0, gsem0)
        issue_idx(1, kb1, ib1, isem1)

        def pair(i, carry):
            wait_idx(kb1, ib1, isem1)
            wait_gather(abuf0, bbuf0, gsem0)
            issue_gather(kb1, ib1, abuf1, bbuf1, gsem1)

            @pl.when(2 * i + 2 < NCH)
            def _i0():
                issue_idx(2 * i + 2, kb0, ib0, isem0)

            @pl.when(i > 0)
            def _w0():
                wait_write(hbuf0, wsem0)

            compute(abuf0, bbuf0, hbuf0)
            issue_write(2 * i, hbuf0, wsem0)

            wait_gather(abuf1, bbuf1, gsem1)

            @pl.when(2 * i + 2 < NCH)
            def _g0():
                wait_idx(kb0, ib0, isem0)
                issue_gather(kb0, ib0, abuf0, bbuf0, gsem0)

            @pl.when(2 * i + 3 < NCH)
            def _i1():
                issue_idx(2 * i + 3, kb1, ib1, isem1)

            @pl.when(i > 0)
            def _w1():
                wait_write(hbuf1, wsem1)

            compute(abuf1, bbuf1, hbuf1)
            issue_write(2 * i + 1, hbuf1, wsem1)
            return carry

        lax.fori_loop(0, NCH // 2, pair, 0)
        wait_write(hbuf0, wsem0)
        wait_write(hbuf1, wsem1)

    return k1(uxb, k_idx, i_idx)


def _k2_logits(h_in, w1t, b1r, w2r, b2r, E, H, EB):
    grid = E // EB
    rows = EB // 128

    def body(h_ref, w1_ref, b1_ref, w2_ref, b2_ref, out_ref):
        z = jnp.maximum(
            jnp.dot(
                h_ref[...].astype(jnp.bfloat16),
                w1_ref[...].astype(jnp.bfloat16),
                preferred_element_type=F32,
            )
            + b1_ref[...],
            0.0,
        )
        lg = jnp.sum(z * w2_ref[...], axis=1) + b2_ref[0, 0]
        out_ref[...] = lg.reshape(1, rows, 128)

    return pl.pallas_call(
        body,
        grid=(grid,),
        in_specs=[
            pl.BlockSpec((EB, H), lambda g: (g, 0)),
            pl.BlockSpec((H, H), lambda g: (0, 0)),
            pl.BlockSpec((1, H), lambda g: (0, 0)),
            pl.BlockSpec((1, H), lambda g: (0, 0)),
            pl.BlockSpec((1, 1), lambda g: (0, 0)),
        ],
        out_specs=pl.BlockSpec((1, rows, 128), lambda g: (g, 0, 0)),
        out_shape=jax.ShapeDtypeStruct((grid, rows, 128), F32),
    )(h_in, w1t, b1r, w2r, b2r)


def _k3_softmax_stats(logits2d):
    def body(l_ref, t_ref, sinv_ref):
        l = l_ref[...]
        m = jnp.max(l)
        t = jnp.exp(l - m)
        t_ref[...] = t
        sinv_ref[...] = jnp.reshape(1.0 / jnp.sum(t), (1, 1))

    return pl.pallas_call(
        body,
        out_shape=(
            jax.ShapeDtypeStruct(logits2d.shape, F32),
            jax.ShapeDtypeStruct((1, 1), F32),
        ),
    )(logits2d)


def _k5_aggregate(x, k_ext, t1d, nb_flat, NH, D, H, half):
    NPW = NH // NW
    RPC = 64
    NPC = RPC // D
    NCH = NPW // NPC
    assert NCH % 2 == 0

    @functools.partial(
        pl.kernel,
        out_type=jax.ShapeDtypeStruct((NH, H), F32),
        mesh=_sc_mesh(),
        scratch_types=[
            pltpu.VMEM_SHARED((x.shape[0], H), F32),
            pltpu.VMEM((NPW * D,), I32),
            pltpu.VMEM((RPC,), I32),
            pltpu.VMEM((RPC,), I32),
            pltpu.VMEM((RPC, H), F32),
            pltpu.VMEM((RPC, H), F32),
            pltpu.VMEM((RPC,), F32),
            pltpu.VMEM((RPC,), F32),
            pltpu.VMEM((NPW, H), F32),
            pltpu.SemaphoreType.DMA,
            pltpu.SemaphoreType.DMA,
        ],
        compiler_params=pltpu.CompilerParams(needs_layout_passes=False),
    )
    def k5(x_hbm, k_hbm, t_hbm, nb_hbm, out_hbm,
           x_sp, nball, kv0, kv1, rows0, rows1, tvals0, tvals1,
           aggbuf, gsem0, gsem1):
        wid = _wid()
        nbase = pl.multiple_of(wid * NPW, 8)

        @pl.when(lax.axis_index("s") == 0)
        def _stage():
            pltpu.sync_copy(x_hbm, x_sp)

        plsc.subcore_barrier()
        pltpu.sync_copy(
            nb_hbm.at[pl.ds((half * NH + wid * NPW) * D, NPW * D)], nball
        )

        def issue_stage1(c, kv, tvals, gsem):
            co = pl.multiple_of(c * RPC, 8)
            pltpu.async_copy(k_hbm.at[nball.at[pl.ds(co, RPC)]], kv, gsem)
            pltpu.async_copy(t_hbm.at[nball.at[pl.ds(co, RPC)]], tvals, gsem)

        def wait_stage1(kv, tvals, gsem):
            pltpu.make_async_copy(k_hbm.at[pl.ds(0, RPC)], kv, gsem).wait()
            pltpu.make_async_copy(t_hbm.at[pl.ds(0, RPC)], tvals, gsem).wait()

        def issue_rows(kv, rows, gsem):
            pltpu.async_copy(x_sp.at[kv], rows, gsem)

        def wait_rows(rows, gsem):
            pltpu.make_async_copy(x_sp.at[pl.ds(0, RPC)], rows, gsem).wait()

        def compute(c, rows, tvals):
            for n in range(NPC):
                node = c * NPC + n

                def splat(r):
                    return plsc.load_gather(tvals, [jnp.full((L,), r, I32)])

                def red(e, acc):
                    tv = splat(n * D + e)
                    return tuple(
                        acc[j] + rows[n * D + e, pl.ds(L * j, L)] * tv
                        for j in range(H // L)
                    )

                tv0 = splat(n * D)
                acc0 = tuple(
                    rows[n * D, pl.ds(L * j, L)] * tv0 for j in range(H // L)
                )
                acc = lax.fori_loop(1, D, red, acc0)
                for j in range(H // L):
                    aggbuf[node, pl.ds(L * j, L)] = acc[j]

        issue_stage1(0, kv0, tvals0, gsem0)
        wait_stage1(kv0, tvals0, gsem0)
        issue_rows(kv0, rows0, gsem0)
        issue_stage1(1, kv1, tvals1, gsem1)

        def pair(i, carry):
            wait_stage1(kv1, tvals1, gsem1)
            issue_rows(kv1, rows1, gsem1)
            wait_rows(rows0, gsem0)
            compute(2 * i, rows0, tvals0)

            @pl.when(2 * i + 2 < NCH)
            def _s0():
                issue_stage1(2 * i + 2, kv0, tvals0, gsem0)

            wait_rows(rows1, gsem1)
            compute(2 * i + 1, rows1, tvals1)

            @pl.when(2 * i + 2 < NCH)
            def _r0():
                wait_stage1(kv0, tvals0, gsem0)
                issue_rows(kv0, rows0, gsem0)

            @pl.when(2 * i + 3 < NCH)
            def _s1():
                issue_stage1(2 * i + 3, kv1, tvals1, gsem1)

            return carry

        lax.fori_loop(0, NCH // 2, pair, 0)
        pltpu.sync_copy(aggbuf, out_hbm.at[pl.ds(nbase, NPW)])

    return k5(x, k_ext, t1d, nb_flat)


def _k6_tail(x, agg, w0, w1, fc1t, fc1br, fc2t, fc2br, sinv, N, H, OUT, NB):
    grid = N // NB

    def body(x_ref, a_ref, w0_ref, w1_ref, f1_ref, b1_ref, f2_ref, b2_ref,
             s_ref, out_ref):
        x2 = (
            jnp.dot(x_ref[...], w0_ref[...], preferred_element_type=F32)
            + jnp.dot(a_ref[...], w1_ref[...], preferred_element_type=F32)
            * s_ref[0, 0]
        )
        z = jnp.maximum(
            jnp.dot(x2, f1_ref[...], preferred_element_type=F32) + b1_ref[...],
            0.0,
        )
        out_ref[...] = (
            jnp.dot(z, f2_ref[...], preferred_element_type=F32) + b2_ref[...]
        )

    return pl.pallas_call(
        body,
        grid=(grid,),
        in_specs=[
            pl.BlockSpec((NB, H), lambda g: (g, 0)),
            pl.BlockSpec((NB, H), lambda g: (g, 0)),
            pl.BlockSpec((H, H), lambda g: (0, 0)),
            pl.BlockSpec((H, H), lambda g: (0, 0)),
            pl.BlockSpec((H, H), lambda g: (0, 0)),
            pl.BlockSpec((1, H), lambda g: (0, 0)),
            pl.BlockSpec((H, OUT), lambda g: (0, 0)),
            pl.BlockSpec((1, OUT), lambda g: (0, 0)),
            pl.BlockSpec((1, 1), lambda g: (0, 0)),
        ],
        out_specs=pl.BlockSpec((NB, OUT), lambda g: (g, 0)),
        out_shape=jax.ShapeDtypeStruct((N, OUT), F32),
    )(x, agg, w0, w1, fc1t, fc1br, fc2t, fc2br, sinv)


def kernel(u, edge_index, neighbor_all, emb_id,
           att_fc1_w, att_fc1_b, att_fc2_w, att_fc2_b,
           w, fc1_w, fc1_b, fc2_w, fc2_b):
    N, H = u.shape
    E = edge_index.shape[1]
    D = neighbor_all.shape[1]
    OUT = fc2_w.shape[0]

    C = 40
    EB = 2560
    PAD = 8
    NP = -(-N // (NW * 8)) * (NW * 8)

    x = emb_id
    k_idx = edge_index[0]
    i_idx = edge_index[1]
    ux = jnp.concatenate([u, x], axis=1)
    q = jnp.arange(L, dtype=I32)
    qblk = jnp.stack([q, q + L], axis=1).reshape(2 * L)
    qfull = jnp.concatenate([32 * j + qblk for j in range(2 * H // 32)])
    uxb16 = ux[:, qfull].astype(jnp.bfloat16)
    uxb = jax.lax.bitcast_convert_type(
        uxb16.reshape(N, H, 2), jnp.float32
    )

    nb_pad = jnp.concatenate(
        [neighbor_all, jnp.full((NP - N, D), E, dtype=I32)], axis=0
    )
    nb_flat = nb_pad.reshape(NP * D)

    h_in = _k1_edge_features(uxb, k_idx, i_idx, E, H, C)

    logits3d = _k2_logits(
        h_in,
        att_fc1_w.T,
        att_fc1_b.reshape(1, H),
        att_fc2_w.reshape(1, H),
        att_fc2_b.reshape(1, 1),
        E, H, EB=EB,
    )
    logits2d = logits3d.reshape(E // 128, 128)

    t2d, sinv = _k3_softmax_stats(logits2d)
    t1d = jnp.concatenate([t2d.reshape(E), jnp.zeros((PAD,), F32)])
    k_ext = jnp.concatenate([k_idx, jnp.zeros((PAD,), I32)])

    NH = NP // 2
    agg = jnp.concatenate(
        [
            _k5_aggregate(x, k_ext, t1d, nb_flat, NH, D, H, half=0),
            _k5_aggregate(x, k_ext, t1d, nb_flat, NH, D, H, half=1),
        ],
        axis=0,
    )[:N]

    return _k6_tail(
        x, agg, w[0], w[1], fc1_w.T, fc1_b.reshape(1, H),
        fc2_w.T, fc2_b.reshape(1, OUT), sinv, N, H, OUT, NB=400,
    )
